# Initial kernel scaffold; baseline (speedup 1.0000x reference)
import jax
import jax.numpy as jnp
from jax.experimental import pallas as pl


def kernel(x, edge_index, tgt_tags, W_self1, W_neigh1, b1, W_self2, W_neigh2, b2, W_lin1, b_lin1, W_lin2, b_lin2):
    raise NotImplementedError("write your pallas kernel here")


# speedup vs baseline: 1.8339x; 1.8339x over previous
import jax
import jax.numpy as jnp
from jax import lax
from jax.experimental import pallas as pl
from jax.experimental.pallas import tpu as pltpu
from jax.experimental.pallas import tpu_sc as plsc

_N = 10000
_E = 160000
_D = 256
_C = 17
_CP = 128
_NC = 2
_NS = 16
_NT = _NC * _NS
_R = 320
_NPAD = _NT * _R
_CAP = 6400
_SCAN = 2000
_K = 64
_L = 16

_BM = 400
_GRID = _N // _BM

_mesh = plsc.VectorSubcoreMesh(
    core_axis_name="c", subcore_axis_name="s", num_cores=_NC, num_subcores=_NS
)


def _wid():
    return lax.axis_index("s") * _NC + lax.axis_index("c")


def _sc_partition_body(edges_hbm, srcl_hbm, dstl_hbm, cnt_hbm, deg_hbm,
                       srcl_v, dstl_v, chunk_s, chunk_d, cnt_v, deg_v):
    wid = _wid()
    lo = wid * _R
    lane = lax.iota(jnp.int32, _L)
    zero16 = jnp.zeros((_L,), jnp.int32)
    trash16 = jnp.full((_L,), _R, jnp.int32)
    ones16 = jnp.ones((_L,), jnp.float32)
    zf16 = jnp.zeros((_L,), jnp.float32)

    def prefill(i, c):
        srcl_v[pl.ds(i * _L, _L)] = zero16
        dstl_v[pl.ds(i * _L, _L)] = trash16
        return c
    lax.fori_loop(0, (_CAP + _L) // _L, prefill, 0)

    def zdeg(i, c):
        deg_v[pl.ds(i * _L, _L)] = zf16
        return c
    lax.fori_loop(0, (_R * _L + _L) // _L, zdeg, 0)

    def chunk_body(c, cnt):
        pltpu.sync_copy(edges_hbm.at[pl.ds(c * _SCAN, _SCAN)], chunk_s)
        pltpu.sync_copy(edges_hbm.at[pl.ds(_E + c * _SCAN, _SCAN)], chunk_d)

        def vec_body(g, cnt):
            s16 = chunk_s[pl.ds(g * _L, _L)]
            d16 = chunk_d[pl.ds(g * _L, _L)]
            dl = d16 - lo
            m = (dl >= 0) & (dl < _R)
            cum = plsc.cumsum(jnp.where(m, 1, 0))
            cc = jnp.minimum(cnt, _CAP - _L)
            pos = jnp.where(m, cc + cum - 1, _CAP)
            plsc.store_scatter(srcl_v, [pos], s16)
            plsc.store_scatter(dstl_v, [pos], jnp.where(m, dl, _R))
            degidx = jnp.where(m, dl * _L + lane, _R * _L)
            plsc.addupdate_scatter(deg_v, [degidx], ones16)
            return cnt + jnp.max(cum)
        return lax.fori_loop(0, _SCAN // _L, vec_body, cnt)

    cnt = lax.fori_loop(0, _E // _SCAN, chunk_body, jnp.int32(0))
    cnt_v[...] = jnp.full((_L,), 1, jnp.int32) * cnt
    pltpu.sync_copy(srcl_v.at[pl.ds(0, _CAP)], srcl_hbm.at[wid])
    pltpu.sync_copy(dstl_v.at[pl.ds(0, _CAP)], dstl_hbm.at[wid])
    pltpu.sync_copy(cnt_v, cnt_hbm.at[wid])
    pltpu.sync_copy(deg_v.at[pl.ds(0, _R * _L)],
                    deg_hbm.at[pl.ds(wid * _R * _L, _R * _L)])


def _sc_partition(edge_index):
    f = pl.kernel(
        _sc_partition_body,
        out_type=[
            jax.ShapeDtypeStruct((_NT, _CAP), jnp.int32),
            jax.ShapeDtypeStruct((_NT, _CAP), jnp.int32),
            jax.ShapeDtypeStruct((_NT, _L), jnp.int32),
            jax.ShapeDtypeStruct((_NPAD * _L,), jnp.float32),
        ],
        mesh=_mesh,
        compiler_params=pltpu.CompilerParams(needs_layout_passes=False),
        scratch_types=[
            pltpu.VMEM((_CAP + _L,), jnp.int32),
            pltpu.VMEM((_CAP + _L,), jnp.int32),
            pltpu.VMEM((_SCAN,), jnp.int32),
            pltpu.VMEM((_SCAN,), jnp.int32),
            pltpu.VMEM((_L,), jnp.int32),
            pltpu.VMEM((_R * _L + _L,), jnp.float32),
        ],
    )
    return f(edge_index.reshape(2 * _E))


def _sc_agg_body(z_hbm, srcl_hbm, dstl_hbm, cnt_hbm, agg_hbm,
                 srcl_v, dstl_v, cnt_v, acc_v, rows0, rows1, sem0, sem1):
    wid = _wid()
    lane = lax.iota(jnp.int32, _L)
    pltpu.sync_copy(srcl_hbm.at[wid], srcl_v)
    pltpu.sync_copy(dstl_hbm.at[wid], dstl_v)
    pltpu.sync_copy(cnt_hbm.at[wid], cnt_v)

    zf16 = jnp.zeros((_L,), jnp.float32)

    def zacc(i, c):
        acc_v[pl.ds(i * _L, _L)] = zf16
        return c
    lax.fori_loop(0, (_R + 1) * _D // _L, zacc, 0)

    n = jnp.max(cnt_v[...])
    n = jnp.minimum(jnp.maximum(n, 1), _CAP)
    nch = (n + _K - 1) // _K

    def start(ci, rows, sem):
        pltpu.async_copy(z_hbm.at[srcl_v.at[pl.ds(ci * _K, _K)]], rows, sem)

    def wait(ci, rows, sem):
        pltpu.make_async_copy(
            z_hbm.at[srcl_v.at[pl.ds(ci * _K, _K)]], rows, sem).wait()

    def process(ebase, rows):
        def grp(q, c):
            off16 = dstl_v[pl.ds(ebase + q * _L, _L)] * _D
            for k in range(_L):
                off = jnp.sum(jnp.where(lane == k, off16, 0))
                ridx = jnp.full((_L,), q * _L + k, jnp.int32)
                for j in range(_D // _L):
                    vals = plsc.load_gather(rows, [ridx, j * _L + lane])
                    plsc.addupdate(acc_v.at[pl.ds(off + j * _L, _L)], vals)
            return c
        lax.fori_loop(0, _K // _L, grp, 0)

    start(0, rows0, sem0)
    npairs = (nch + 1) // 2

    def pair(p, c):
        c0 = p * 2

        @pl.when(c0 + 1 < nch)
        def _():
            start(c0 + 1, rows1, sem1)
        wait(c0, rows0, sem0)
        process(c0 * _K, rows0)

        @pl.when(c0 + 2 < nch)
        def _():
            start(c0 + 2, rows0, sem0)

        @pl.when(c0 + 1 < nch)
        def _():
            wait(c0 + 1, rows1, sem1)
            process((c0 + 1) * _K, rows1)
        return c
    lax.fori_loop(0, npairs, pair, 0)
    pltpu.sync_copy(acc_v.at[pl.ds(0, _R * _D)],
                    agg_hbm.at[pl.ds(wid * _R * _D, _R * _D)])


def _sc_agg(z, srcl, dstl, cnt):
    f = pl.kernel(
        _sc_agg_body,
        out_type=jax.ShapeDtypeStruct((_NPAD * _D,), jnp.float32),
        mesh=_mesh,
        compiler_params=pltpu.CompilerParams(needs_layout_passes=False),
        scratch_types=[
            pltpu.VMEM((_CAP,), jnp.int32),
            pltpu.VMEM((_CAP,), jnp.int32),
            pltpu.VMEM((_L,), jnp.int32),
            pltpu.VMEM(((_R + 1) * _D,), jnp.float32),
            pltpu.VMEM((_K, _D), jnp.float32),
            pltpu.VMEM((_K, _D), jnp.float32),
            pltpu.SemaphoreType.DMA,
            pltpu.SemaphoreType.DMA,
        ],
    )
    return f(z, srcl, dstl, cnt)


def _elu(v):
    return jnp.where(v > 0, v, jnp.exp(jnp.minimum(v, 0.0)) - 1.0)


def _tc1_body(x_ref, agg_ref, deg_ref, ws_ref, wn_ref, b_ref, h_ref):
    deg = jnp.sum(deg_ref[...], axis=1, keepdims=True)
    degm = jnp.maximum(deg, 1.0)
    aggd = agg_ref[...] / degm
    h_ref[...] = _elu(
        jnp.dot(x_ref[...], ws_ref[...], preferred_element_type=jnp.float32)
        + jnp.dot(aggd, wn_ref[...], preferred_element_type=jnp.float32)
        + b_ref[...])


def _tc1(x, agg1, deg16, W_self1, W_neigh1, b1):
    return pl.pallas_call(
        _tc1_body,
        grid=(_GRID,),
        in_specs=[
            pl.BlockSpec((_BM, _D), lambda i: (i, 0)),
            pl.BlockSpec((_BM, _D), lambda i: (i, 0)),
            pl.BlockSpec((_BM, _L), lambda i: (i, 0)),
            pl.BlockSpec((_D, _D), lambda i: (0, 0)),
            pl.BlockSpec((_D, _D), lambda i: (0, 0)),
            pl.BlockSpec((1, _D), lambda i: (0, 0)),
        ],
        out_specs=pl.BlockSpec((_BM, _D), lambda i: (i, 0)),
        out_shape=jax.ShapeDtypeStruct((_N, _D), jnp.float32),
    )(x, agg1, deg16, W_self1, W_neigh1, b1)


def _tc2_body(h1_ref, agg_ref, deg_ref, tgt_ref, ws_ref, wn_ref, b_ref,
              wl1_ref, bl1_ref, wl2_ref, bl2_ref, loss_ref, pred_ref):
    i = pl.program_id(0)
    deg = jnp.sum(deg_ref[...], axis=1, keepdims=True)
    degm = jnp.maximum(deg, 1.0)
    aggd = agg_ref[...] / degm
    h2 = _elu(
        jnp.dot(h1_ref[...], ws_ref[...], preferred_element_type=jnp.float32)
        + jnp.dot(aggd, wn_ref[...], preferred_element_type=jnp.float32)
        + b_ref[...])
    h3 = _elu(jnp.dot(h2, wl1_ref[...], preferred_element_type=jnp.float32)
              + bl1_ref[...])
    lg = (jnp.dot(h3, wl2_ref[...], preferred_element_type=jnp.float32)
          + bl2_ref[...])
    m = jnp.max(lg, axis=1, keepdims=True)
    ex = jnp.exp(lg - m)
    lse = m + jnp.log(jnp.sum(ex, axis=1, keepdims=True))
    lanes = lax.broadcasted_iota(jnp.int32, (_BM, _CP), 1)
    picked = jnp.sum(jnp.where(tgt_ref[...] == lanes, lg, 0.0),
                     axis=1, keepdims=True)
    part = jnp.reshape(jnp.sum(lse - picked), (1, 1))

    @pl.when(i == 0)
    def _():
        loss_ref[...] = jnp.zeros((1, 1), jnp.float32)
    loss_ref[...] += part
    ismax = lg == m
    idx = jnp.min(jnp.where(ismax, lanes, _CP), axis=1, keepdims=True)
    pred_ref[...] = jnp.broadcast_to(idx, (_BM, _CP))


def _tc2(h1, agg2, deg16, tgt128, W_self2, W_neigh2, b2,
         W_lin1, b_lin1, W_lin2p, b_lin2p):
    return pl.pallas_call(
        _tc2_body,
        grid=(_GRID,),
        in_specs=[
            pl.BlockSpec((_BM, _D), lambda i: (i, 0)),
            pl.BlockSpec((_BM, _D), lambda i: (i, 0)),
            pl.BlockSpec((_BM, _L), lambda i: (i, 0)),
            pl.BlockSpec((_BM, _CP), lambda i: (i, 0)),
            pl.BlockSpec((_D, _D), lambda i: (0, 0)),
            pl.BlockSpec((_D, _D), lambda i: (0, 0)),
            pl.BlockSpec((1, _D), lambda i: (0, 0)),
            pl.BlockSpec((_D, _D), lambda i: (0, 0)),
            pl.BlockSpec((1, _D), lambda i: (0, 0)),
            pl.BlockSpec((_D, _CP), lambda i: (0, 0)),
            pl.BlockSpec((1, _CP), lambda i: (0, 0)),
        ],
        out_specs=[
            pl.BlockSpec((1, 1), lambda i: (0, 0)),
            pl.BlockSpec((_BM, _CP), lambda i: (i, 0)),
        ],
        out_shape=[
            jax.ShapeDtypeStruct((1, 1), jnp.float32),
            jax.ShapeDtypeStruct((_N, _CP), jnp.int32),
        ],
    )(h1, agg2, deg16, tgt128, W_self2, W_neigh2, b2,
      W_lin1, b_lin1, W_lin2p, b_lin2p)


def kernel(x, edge_index, tgt_tags,
           W_self1, W_neigh1, b1,
           W_self2, W_neigh2, b2,
           W_lin1, b_lin1, W_lin2, b_lin2):
    srcl, dstl, cnt, deg_flat = _sc_partition(edge_index)
    deg16 = deg_flat.reshape(_NPAD, _L)

    agg1 = _sc_agg(x, srcl, dstl, cnt).reshape(_NPAD, _D)
    h1 = _tc1(x, agg1, deg16, W_self1, W_neigh1, b1.reshape(1, _D))
    agg2 = _sc_agg(h1, srcl, dstl, cnt).reshape(_NPAD, _D)

    W_lin2p = jnp.concatenate(
        [W_lin2, jnp.zeros((_D, _CP - _C), jnp.float32)], axis=1)
    b_lin2p = jnp.concatenate(
        [b_lin2, jnp.full((_CP - _C,), -1e9, jnp.float32)]).reshape(1, _CP)
    tgt128 = jnp.broadcast_to(tgt_tags[:, None], (_N, _CP))

    loss11, pred128 = _tc2(h1, agg2, deg16, tgt128,
                           W_self2, W_neigh2, b2.reshape(1, _D),
                           W_lin1, b_lin1.reshape(1, _D), W_lin2p, b_lin2p)
    loss = loss11[0, 0] / jnp.float32(_N)
    pred = pred128[:, 0]
    return (loss, pred)


# speedup vs baseline: 2.0335x; 1.1088x over previous
import jax
import jax.numpy as jnp
from jax import lax
from jax.experimental import pallas as pl
from jax.experimental.pallas import tpu as pltpu
from jax.experimental.pallas import tpu_sc as plsc

_N = 10000
_E = 160000
_D = 256
_C = 17
_CP = 128
_NC = 2
_NS = 16
_NT = _NC * _NS
_R = 320
_NPAD = _NT * _R
_CAP = 6400
_SCAN = 4000
_K = 64
_L = 16

_BM = 400
_GRID = _N // _BM

_mesh = plsc.VectorSubcoreMesh(
    core_axis_name="c", subcore_axis_name="s", num_cores=_NC, num_subcores=_NS
)


def _wid():
    return lax.axis_index("s") * _NC + lax.axis_index("c")


def _sc_partition_body(edges_hbm, srcl_hbm, dstl_hbm, cnt_hbm, deg_hbm,
                       srcl_v, dstl_v, cs_a, cd_a, cs_b, cd_b, cnt_v, deg_v,
                       s0, s1, s2, s3):
    wid = _wid()
    lo = wid * _R
    _NCH = _E // _SCAN

    def prefill(i, c):
        srcl_v[pl.ds(i * _L, _L)] = jnp.zeros((_L,), jnp.int32)
        dstl_v[pl.ds(i * _L, _L)] = jnp.full((_L,), _R, jnp.int32)
        return c
    lax.fori_loop(0, (_CAP + _L) // _L, prefill, 0)

    def zdeg(i, c):
        deg_v[pl.ds(i * _L, _L)] = jnp.zeros((_L,), jnp.float32)
        return c
    lax.fori_loop(0, (_R * _L + _L) // _L, zdeg, 0)

    def startc(c, cs, cd, ss, sd):
        pltpu.async_copy(edges_hbm.at[pl.ds(c * _SCAN, _SCAN)], cs, ss)
        pltpu.async_copy(edges_hbm.at[pl.ds(_E + c * _SCAN, _SCAN)], cd, sd)

    def waitc(c, cs, cd, ss, sd):
        pltpu.make_async_copy(
            edges_hbm.at[pl.ds(c * _SCAN, _SCAN)], cs, ss).wait()
        pltpu.make_async_copy(
            edges_hbm.at[pl.ds(_E + c * _SCAN, _SCAN)], cd, sd).wait()

    def scan_chunk(cs, cd, cntv):
        def vec_body(g, cntv):
            lane = lax.iota(jnp.int32, _L)
            ones16 = jnp.ones((_L,), jnp.float32)
            out = cntv
            for u in range(2):
                s16 = cs[pl.ds((g * 2 + u) * _L, _L)]
                d16 = cd[pl.ds((g * 2 + u) * _L, _L)]
                dl = d16 - lo
                m = (dl >= 0) & (dl < _R)
                cum = plsc.cumsum(jnp.where(m, 1, 0))
                pos = jnp.where(
                    m, jnp.minimum(out, _CAP - _L) + cum - 1, _CAP)
                plsc.store_scatter(srcl_v, [pos], s16)
                plsc.store_scatter(dstl_v, [pos], jnp.where(m, dl, _R))
                degidx = jnp.where(m, dl * _L + lane, _R * _L)
                plsc.addupdate_scatter(deg_v, [degidx], ones16)
                out = out + plsc.all_reduce_population_count(m)
            return out
        return lax.fori_loop(0, _SCAN // _L // 2, vec_body, cntv)

    startc(0, cs_a, cd_a, s0, s1)

    def pair(p, cntv):
        c0 = p * 2
        startc(c0 + 1, cs_b, cd_b, s2, s3)
        waitc(c0, cs_a, cd_a, s0, s1)
        cntv = scan_chunk(cs_a, cd_a, cntv)

        @pl.when(c0 + 2 < _NCH)
        def _():
            startc(c0 + 2, cs_a, cd_a, s0, s1)
        waitc(c0 + 1, cs_b, cd_b, s2, s3)
        cntv = scan_chunk(cs_b, cd_b, cntv)
        return cntv
    cntv = lax.fori_loop(0, _NCH // 2, pair, jnp.zeros((_L,), jnp.int32))
    cnt_v[...] = cntv
    pltpu.sync_copy(srcl_v.at[pl.ds(0, _CAP)], srcl_hbm.at[wid])
    pltpu.sync_copy(dstl_v.at[pl.ds(0, _CAP)], dstl_hbm.at[wid])
    pltpu.sync_copy(cnt_v, cnt_hbm.at[wid])
    pltpu.sync_copy(deg_v.at[pl.ds(0, _R * _L)],
                    deg_hbm.at[pl.ds(wid * _R * _L, _R * _L)])


def _sc_partition(edge_index):
    f = pl.kernel(
        _sc_partition_body,
        out_type=[
            jax.ShapeDtypeStruct((_NT, _CAP), jnp.int32),
            jax.ShapeDtypeStruct((_NT, _CAP), jnp.int32),
            jax.ShapeDtypeStruct((_NT, _L), jnp.int32),
            jax.ShapeDtypeStruct((_NPAD * _L,), jnp.float32),
        ],
        mesh=_mesh,
        compiler_params=pltpu.CompilerParams(needs_layout_passes=False),
        scratch_types=[
            pltpu.VMEM((_CAP + _L,), jnp.int32),
            pltpu.VMEM((_CAP + _L,), jnp.int32),
            pltpu.VMEM((_SCAN,), jnp.int32),
            pltpu.VMEM((_SCAN,), jnp.int32),
            pltpu.VMEM((_SCAN,), jnp.int32),
            pltpu.VMEM((_SCAN,), jnp.int32),
            pltpu.VMEM((_L,), jnp.int32),
            pltpu.VMEM((_R * _L + _L,), jnp.float32),
            pltpu.SemaphoreType.DMA,
            pltpu.SemaphoreType.DMA,
            pltpu.SemaphoreType.DMA,
            pltpu.SemaphoreType.DMA,
        ],
    )
    return f(edge_index.reshape(2 * _E))


def _sc_agg_body(z_hbm, srcl_hbm, dstl_hbm, cnt_hbm, agg_hbm,
                 srcl_v, dstl_v, cnt_v, acc_v, rows0, rows1, sem0, sem1):
    wid = _wid()
    lane = lax.iota(jnp.int32, _L)
    pltpu.sync_copy(srcl_hbm.at[wid], srcl_v)
    pltpu.sync_copy(dstl_hbm.at[wid], dstl_v)
    pltpu.sync_copy(cnt_hbm.at[wid], cnt_v)

    zf16 = jnp.zeros((_L,), jnp.float32)

    def zacc(i, c):
        acc_v[pl.ds(i * _L, _L)] = zf16
        return c
    lax.fori_loop(0, (_R + 1) * _D // _L, zacc, 0)

    n = jnp.max(cnt_v[...])
    n = jnp.minimum(jnp.maximum(n, 1), _CAP)
    nch = (n + _K - 1) // _K

    def start(ci, rows, sem):
        pltpu.async_copy(z_hbm.at[srcl_v.at[pl.ds(ci * _K, _K)]], rows, sem)

    def wait(ci, rows, sem):
        pltpu.make_async_copy(
            z_hbm.at[srcl_v.at[pl.ds(ci * _K, _K)]], rows, sem).wait()

    def process(ebase, rows):
        def grp(q, c):
            off16 = dstl_v[pl.ds(ebase + q * _L, _L)] * _D
            for k in range(_L):
                off = jnp.sum(jnp.where(lane == k, off16, 0))
                ridx = jnp.full((_L,), q * _L + k, jnp.int32)
                for j in range(_D // _L):
                    vals = plsc.load_gather(rows, [ridx, j * _L + lane])
                    plsc.addupdate(acc_v.at[pl.ds(off + j * _L, _L)], vals)
            return c
        lax.fori_loop(0, _K // _L, grp, 0)

    start(0, rows0, sem0)
    npairs = (nch + 1) // 2

    def pair(p, c):
        c0 = p * 2

        @pl.when(c0 + 1 < nch)
        def _():
            start(c0 + 1, rows1, sem1)
        wait(c0, rows0, sem0)
        process(c0 * _K, rows0)

        @pl.when(c0 + 2 < nch)
        def _():
            start(c0 + 2, rows0, sem0)

        @pl.when(c0 + 1 < nch)
        def _():
            wait(c0 + 1, rows1, sem1)
            process((c0 + 1) * _K, rows1)
        return c
    lax.fori_loop(0, npairs, pair, 0)
    pltpu.sync_copy(acc_v.at[pl.ds(0, _R * _D)],
                    agg_hbm.at[pl.ds(wid * _R * _D, _R * _D)])


def _sc_agg(z, srcl, dstl, cnt):
    f = pl.kernel(
        _sc_agg_body,
        out_type=jax.ShapeDtypeStruct((_NPAD * _D,), jnp.float32),
        mesh=_mesh,
        compiler_params=pltpu.CompilerParams(needs_layout_passes=False),
        scratch_types=[
            pltpu.VMEM((_CAP,), jnp.int32),
            pltpu.VMEM((_CAP,), jnp.int32),
            pltpu.VMEM((_L,), jnp.int32),
            pltpu.VMEM(((_R + 1) * _D,), jnp.float32),
            pltpu.VMEM((_K, _D), jnp.float32),
            pltpu.VMEM((_K, _D), jnp.float32),
            pltpu.SemaphoreType.DMA,
            pltpu.SemaphoreType.DMA,
        ],
    )
    return f(z, srcl, dstl, cnt)


def _elu(v):
    return jnp.where(v > 0, v, jnp.exp(jnp.minimum(v, 0.0)) - 1.0)


def _tc1_body(x_ref, agg_ref, deg_ref, ws_ref, wn_ref, b_ref, h_ref):
    deg = jnp.sum(deg_ref[...], axis=1, keepdims=True)
    degm = jnp.maximum(deg, 1.0)
    aggd = agg_ref[...] / degm
    h_ref[...] = _elu(
        jnp.dot(x_ref[...], ws_ref[...], preferred_element_type=jnp.float32)
        + jnp.dot(aggd, wn_ref[...], preferred_element_type=jnp.float32)
        + b_ref[...])


def _tc1(x, agg1, deg16, W_self1, W_neigh1, b1):
    return pl.pallas_call(
        _tc1_body,
        grid=(_GRID,),
        in_specs=[
            pl.BlockSpec((_BM, _D), lambda i: (i, 0)),
            pl.BlockSpec((_BM, _D), lambda i: (i, 0)),
            pl.BlockSpec((_BM, _L), lambda i: (i, 0)),
            pl.BlockSpec((_D, _D), lambda i: (0, 0)),
            pl.BlockSpec((_D, _D), lambda i: (0, 0)),
            pl.BlockSpec((1, _D), lambda i: (0, 0)),
        ],
        out_specs=pl.BlockSpec((_BM, _D), lambda i: (i, 0)),
        out_shape=jax.ShapeDtypeStruct((_N, _D), jnp.float32),
    )(x, agg1, deg16, W_self1, W_neigh1, b1)


def _tc2_body(h1_ref, agg_ref, deg_ref, tgt_ref, ws_ref, wn_ref, b_ref,
              wl1_ref, bl1_ref, wl2_ref, bl2_ref, loss_ref, pred_ref):
    i = pl.program_id(0)
    deg = jnp.sum(deg_ref[...], axis=1, keepdims=True)
    degm = jnp.maximum(deg, 1.0)
    aggd = agg_ref[...] / degm
    h2 = _elu(
        jnp.dot(h1_ref[...], ws_ref[...], preferred_element_type=jnp.float32)
        + jnp.dot(aggd, wn_ref[...], preferred_element_type=jnp.float32)
        + b_ref[...])
    h3 = _elu(jnp.dot(h2, wl1_ref[...], preferred_element_type=jnp.float32)
              + bl1_ref[...])
    lg = (jnp.dot(h3, wl2_ref[...], preferred_element_type=jnp.float32)
          + bl2_ref[...])
    m = jnp.max(lg, axis=1, keepdims=True)
    ex = jnp.exp(lg - m)
    lse = m + jnp.log(jnp.sum(ex, axis=1, keepdims=True))
    lanes = lax.broadcasted_iota(jnp.int32, (_BM, _CP), 1)
    picked = jnp.sum(jnp.where(tgt_ref[...] == lanes, lg, 0.0),
                     axis=1, keepdims=True)
    part = jnp.reshape(jnp.sum(lse - picked), (1, 1))

    @pl.when(i == 0)
    def _():
        loss_ref[...] = jnp.zeros((1, 1), jnp.float32)
    loss_ref[...] += part
    ismax = lg == m
    idx = jnp.min(jnp.where(ismax, lanes, _CP), axis=1, keepdims=True)
    pred_ref[...] = jnp.broadcast_to(idx, (_BM, _CP))


def _tc2(h1, agg2, deg16, tgt128, W_self2, W_neigh2, b2,
         W_lin1, b_lin1, W_lin2p, b_lin2p):
    return pl.pallas_call(
        _tc2_body,
        grid=(_GRID,),
        in_specs=[
            pl.BlockSpec((_BM, _D), lambda i: (i, 0)),
            pl.BlockSpec((_BM, _D), lambda i: (i, 0)),
            pl.BlockSpec((_BM, _L), lambda i: (i, 0)),
            pl.BlockSpec((_BM, _CP), lambda i: (i, 0)),
            pl.BlockSpec((_D, _D), lambda i: (0, 0)),
            pl.BlockSpec((_D, _D), lambda i: (0, 0)),
            pl.BlockSpec((1, _D), lambda i: (0, 0)),
            pl.BlockSpec((_D, _D), lambda i: (0, 0)),
            pl.BlockSpec((1, _D), lambda i: (0, 0)),
            pl.BlockSpec((_D, _CP), lambda i: (0, 0)),
            pl.BlockSpec((1, _CP), lambda i: (0, 0)),
        ],
        out_specs=[
            pl.BlockSpec((1, 1), lambda i: (0, 0)),
            pl.BlockSpec((_BM, _CP), lambda i: (i, 0)),
        ],
        out_shape=[
            jax.ShapeDtypeStruct((1, 1), jnp.float32),
            jax.ShapeDtypeStruct((_N, _CP), jnp.int32),
        ],
    )(h1, agg2, deg16, tgt128, W_self2, W_neigh2, b2,
      W_lin1, b_lin1, W_lin2p, b_lin2p)


def kernel(x, edge_index, tgt_tags,
           W_self1, W_neigh1, b1,
           W_self2, W_neigh2, b2,
           W_lin1, b_lin1, W_lin2, b_lin2):
    srcl, dstl, cnt, deg_flat = _sc_partition(edge_index)
    deg16 = deg_flat.reshape(_NPAD, _L)

    agg1 = _sc_agg(x, srcl, dstl, cnt).reshape(_NPAD, _D)
    h1 = _tc1(x, agg1, deg16, W_self1, W_neigh1, b1.reshape(1, _D))
    agg2 = _sc_agg(h1, srcl, dstl, cnt).reshape(_NPAD, _D)

    W_lin2p = jnp.concatenate(
        [W_lin2, jnp.zeros((_D, _CP - _C), jnp.float32)], axis=1)
    b_lin2p = jnp.concatenate(
        [b_lin2, jnp.full((_CP - _C,), -1e9, jnp.float32)]).reshape(1, _CP)
    tgt128 = jnp.broadcast_to(tgt_tags[:, None], (_N, _CP))

    loss11, pred128 = _tc2(h1, agg2, deg16, tgt128,
                           W_self2, W_neigh2, b2.reshape(1, _D),
                           W_lin1, b_lin1.reshape(1, _D), W_lin2p, b_lin2p)
    loss = loss11[0, 0] / jnp.float32(_N)
    pred = pred128[:, 0]
    return (loss, pred)


# speedup vs baseline: 2.1539x; 1.0592x over previous
import jax
import jax.numpy as jnp
from jax import lax
from jax.experimental import pallas as pl
from jax.experimental.pallas import tpu as pltpu
from jax.experimental.pallas import tpu_sc as plsc

_N = 10000
_E = 160000
_D = 256
_C = 17
_CP = 128
_NC = 2
_NS = 16
_NT = _NC * _NS
_R = 320
_NPAD = _NT * _R
_CAP = 6400
_SCAN = 4000
_K = 64
_L = 16

_BM = 400
_GRID = _N // _BM

_mesh = plsc.VectorSubcoreMesh(
    core_axis_name="c", subcore_axis_name="s", num_cores=_NC, num_subcores=_NS
)


def _wid():
    return lax.axis_index("s") * _NC + lax.axis_index("c")


def _sc_partition_body(edges_hbm, srcl_hbm, dstl_hbm, cnt_hbm, deg_hbm,
                       srcl_v, dstl_v, cs_a, cd_a, cs_b, cd_b, cnt_v, deg_v,
                       s0, s1, s2, s3):
    wid = _wid()
    lo = wid * _R
    _NCH = _E // _SCAN

    def prefill(i, c):
        srcl_v[pl.ds(i * _L, _L)] = jnp.zeros((_L,), jnp.int32)
        dstl_v[pl.ds(i * _L, _L)] = jnp.full((_L,), _R, jnp.int32)
        return c
    lax.fori_loop(0, (_CAP + _L) // _L, prefill, 0)

    def zdeg(i, c):
        deg_v[pl.ds(i * _L, _L)] = jnp.zeros((_L,), jnp.float32)
        return c
    lax.fori_loop(0, (_R * _L + _L) // _L, zdeg, 0)

    def startc(c, cs, cd, ss, sd):
        pltpu.async_copy(edges_hbm.at[pl.ds(c * _SCAN, _SCAN)], cs, ss)
        pltpu.async_copy(edges_hbm.at[pl.ds(_E + c * _SCAN, _SCAN)], cd, sd)

    def waitc(c, cs, cd, ss, sd):
        pltpu.make_async_copy(
            edges_hbm.at[pl.ds(c * _SCAN, _SCAN)], cs, ss).wait()
        pltpu.make_async_copy(
            edges_hbm.at[pl.ds(_E + c * _SCAN, _SCAN)], cd, sd).wait()

    def scan_chunk(cs, cd, cntv):
        def vec_body(g, cntv):
            lane = lax.iota(jnp.int32, _L)
            ones16 = jnp.ones((_L,), jnp.float32)
            out = cntv
            for u in range(2):
                s16 = cs[pl.ds((g * 2 + u) * _L, _L)]
                d16 = cd[pl.ds((g * 2 + u) * _L, _L)]
                dl = d16 - lo
                m = (dl >= 0) & (dl < _R)
                cum = plsc.cumsum(jnp.where(m, 1, 0))
                pos = jnp.where(
                    m, jnp.minimum(out, _CAP - _L) + cum - 1, _CAP)
                plsc.store_scatter(srcl_v, [pos], s16)
                plsc.store_scatter(dstl_v, [pos], jnp.where(m, dl, _R))
                degidx = jnp.where(m, dl * _L + lane, _R * _L)
                plsc.addupdate_scatter(deg_v, [degidx], ones16)
                out = out + plsc.all_reduce_population_count(m)
            return out
        return lax.fori_loop(0, _SCAN // _L // 2, vec_body, cntv)

    startc(0, cs_a, cd_a, s0, s1)

    def pair(p, cntv):
        c0 = p * 2
        startc(c0 + 1, cs_b, cd_b, s2, s3)
        waitc(c0, cs_a, cd_a, s0, s1)
        cntv = scan_chunk(cs_a, cd_a, cntv)

        @pl.when(c0 + 2 < _NCH)
        def _():
            startc(c0 + 2, cs_a, cd_a, s0, s1)
        waitc(c0 + 1, cs_b, cd_b, s2, s3)
        cntv = scan_chunk(cs_b, cd_b, cntv)
        return cntv
    cntv = lax.fori_loop(0, _NCH // 2, pair, jnp.zeros((_L,), jnp.int32))
    cnt_v[...] = cntv
    pltpu.sync_copy(srcl_v.at[pl.ds(0, _CAP)], srcl_hbm.at[wid])
    pltpu.sync_copy(dstl_v.at[pl.ds(0, _CAP)], dstl_hbm.at[wid])
    pltpu.sync_copy(cnt_v, cnt_hbm.at[wid])
    pltpu.sync_copy(deg_v.at[pl.ds(0, _R * _L)],
                    deg_hbm.at[pl.ds(wid * _R * _L, _R * _L)])


def _sc_partition(edge_index):
    f = pl.kernel(
        _sc_partition_body,
        out_type=[
            jax.ShapeDtypeStruct((_NT, _CAP), jnp.int32),
            jax.ShapeDtypeStruct((_NT, _CAP), jnp.int32),
            jax.ShapeDtypeStruct((_NT, _L), jnp.int32),
            jax.ShapeDtypeStruct((_NPAD * _L,), jnp.float32),
        ],
        mesh=_mesh,
        compiler_params=pltpu.CompilerParams(needs_layout_passes=False),
        scratch_types=[
            pltpu.VMEM((_CAP + _L,), jnp.int32),
            pltpu.VMEM((_CAP + _L,), jnp.int32),
            pltpu.VMEM((_SCAN,), jnp.int32),
            pltpu.VMEM((_SCAN,), jnp.int32),
            pltpu.VMEM((_SCAN,), jnp.int32),
            pltpu.VMEM((_SCAN,), jnp.int32),
            pltpu.VMEM((_L,), jnp.int32),
            pltpu.VMEM((_R * _L + _L,), jnp.float32),
            pltpu.SemaphoreType.DMA,
            pltpu.SemaphoreType.DMA,
            pltpu.SemaphoreType.DMA,
            pltpu.SemaphoreType.DMA,
        ],
    )
    return f(edge_index.reshape(2 * _E))


def _sc_agg_body(z_hbm, srcl_hbm, dstl_hbm, cnt_hbm, agg_hbm,
                 srcl_v, dstl_v, cnt_v, acc_v, rows0, rows1, sem0, sem1):
    wid = _wid()
    lane = lax.iota(jnp.int32, _L)
    pltpu.sync_copy(srcl_hbm.at[wid], srcl_v)
    pltpu.sync_copy(dstl_hbm.at[wid], dstl_v)
    pltpu.sync_copy(cnt_hbm.at[wid], cnt_v)

    zf16 = jnp.zeros((_L,), jnp.float32)

    def zacc(i, c):
        acc_v[pl.ds(i * _L, _L)] = zf16
        return c
    lax.fori_loop(0, (_R + 1) * _D // _L, zacc, 0)

    n = jnp.max(cnt_v[...])
    n = jnp.minimum(jnp.maximum(n, 1), _CAP)
    nch = (n + _K - 1) // _K

    def start(ci, rows, sem):
        pltpu.async_copy(z_hbm.at[srcl_v.at[pl.ds(ci * _K, _K)]], rows, sem)

    def wait(ci, rows, sem):
        pltpu.make_async_copy(
            z_hbm.at[srcl_v.at[pl.ds(ci * _K, _K)]], rows, sem).wait()

    def process(ebase, rows):
        def grp(q, c):
            lane_i = lax.iota(jnp.int32, _L)
            off16 = dstl_v[pl.ds(ebase + q * _L, _L)] * _D
            for k in range(_L):
                off = jnp.sum(jnp.where(lane_i == k, off16, 0))
                e = q * _L + k
                for j in range(_D // _L):
                    plsc.addupdate(acc_v.at[pl.ds(off + j * _L, _L)],
                                   rows[e, pl.ds(j * _L, _L)])
            return c
        lax.fori_loop(0, _K // _L, grp, 0)

    start(0, rows0, sem0)
    npairs = (nch + 1) // 2

    def pair(p, c):
        c0 = p * 2

        @pl.when(c0 + 1 < nch)
        def _():
            start(c0 + 1, rows1, sem1)
        wait(c0, rows0, sem0)
        process(c0 * _K, rows0)

        @pl.when(c0 + 2 < nch)
        def _():
            start(c0 + 2, rows0, sem0)

        @pl.when(c0 + 1 < nch)
        def _():
            wait(c0 + 1, rows1, sem1)
            process((c0 + 1) * _K, rows1)
        return c
    lax.fori_loop(0, npairs, pair, 0)
    pltpu.sync_copy(acc_v.at[pl.ds(0, _R * _D)],
                    agg_hbm.at[pl.ds(wid * _R * _D, _R * _D)])


def _sc_agg(z, srcl, dstl, cnt):
    f = pl.kernel(
        _sc_agg_body,
        out_type=jax.ShapeDtypeStruct((_NPAD * _D,), jnp.float32),
        mesh=_mesh,
        compiler_params=pltpu.CompilerParams(needs_layout_passes=False),
        scratch_types=[
            pltpu.VMEM((_CAP,), jnp.int32),
            pltpu.VMEM((_CAP,), jnp.int32),
            pltpu.VMEM((_L,), jnp.int32),
            pltpu.VMEM(((_R + 1) * _D,), jnp.float32),
            pltpu.VMEM((_K, _D), jnp.float32),
            pltpu.VMEM((_K, _D), jnp.float32),
            pltpu.SemaphoreType.DMA,
            pltpu.SemaphoreType.DMA,
        ],
    )
    return f(z, srcl, dstl, cnt)


def _elu(v):
    return jnp.where(v > 0, v, jnp.exp(jnp.minimum(v, 0.0)) - 1.0)


def _tc1_body(x_ref, agg_ref, deg_ref, ws_ref, wn_ref, b_ref, h_ref):
    deg = jnp.sum(deg_ref[...], axis=1, keepdims=True)
    degm = jnp.maximum(deg, 1.0)
    aggd = agg_ref[...] / degm
    h_ref[...] = _elu(
        jnp.dot(x_ref[...], ws_ref[...], preferred_element_type=jnp.float32)
        + jnp.dot(aggd, wn_ref[...], preferred_element_type=jnp.float32)
        + b_ref[...])


def _tc1(x, agg1, deg16, W_self1, W_neigh1, b1):
    return pl.pallas_call(
        _tc1_body,
        grid=(_GRID,),
        in_specs=[
            pl.BlockSpec((_BM, _D), lambda i: (i, 0)),
            pl.BlockSpec((_BM, _D), lambda i: (i, 0)),
            pl.BlockSpec((_BM, _L), lambda i: (i, 0)),
            pl.BlockSpec((_D, _D), lambda i: (0, 0)),
            pl.BlockSpec((_D, _D), lambda i: (0, 0)),
            pl.BlockSpec((1, _D), lambda i: (0, 0)),
        ],
        out_specs=pl.BlockSpec((_BM, _D), lambda i: (i, 0)),
        out_shape=jax.ShapeDtypeStruct((_N, _D), jnp.float32),
    )(x, agg1, deg16, W_self1, W_neigh1, b1)


def _tc2_body(h1_ref, agg_ref, deg_ref, tgt_ref, ws_ref, wn_ref, b_ref,
              wl1_ref, bl1_ref, wl2_ref, bl2_ref, loss_ref, pred_ref):
    i = pl.program_id(0)
    deg = jnp.sum(deg_ref[...], axis=1, keepdims=True)
    degm = jnp.maximum(deg, 1.0)
    aggd = agg_ref[...] / degm
    h2 = _elu(
        jnp.dot(h1_ref[...], ws_ref[...], preferred_element_type=jnp.float32)
        + jnp.dot(aggd, wn_ref[...], preferred_element_type=jnp.float32)
        + b_ref[...])
    h3 = _elu(jnp.dot(h2, wl1_ref[...], preferred_element_type=jnp.float32)
              + bl1_ref[...])
    lg = (jnp.dot(h3, wl2_ref[...], preferred_element_type=jnp.float32)
          + bl2_ref[...])
    m = jnp.max(lg, axis=1, keepdims=True)
    ex = jnp.exp(lg - m)
    lse = m + jnp.log(jnp.sum(ex, axis=1, keepdims=True))
    lanes = lax.broadcasted_iota(jnp.int32, (_BM, _CP), 1)
    picked = jnp.sum(jnp.where(tgt_ref[...] == lanes, lg, 0.0),
                     axis=1, keepdims=True)
    part = jnp.reshape(jnp.sum(lse - picked), (1, 1))

    @pl.when(i == 0)
    def _():
        loss_ref[...] = jnp.zeros((1, 1), jnp.float32)
    loss_ref[...] += part
    ismax = lg == m
    idx = jnp.min(jnp.where(ismax, lanes, _CP), axis=1, keepdims=True)
    pred_ref[...] = jnp.broadcast_to(idx, (_BM, _CP))


def _tc2(h1, agg2, deg16, tgt128, W_self2, W_neigh2, b2,
         W_lin1, b_lin1, W_lin2p, b_lin2p):
    return pl.pallas_call(
        _tc2_body,
        grid=(_GRID,),
        in_specs=[
            pl.BlockSpec((_BM, _D), lambda i: (i, 0)),
            pl.BlockSpec((_BM, _D), lambda i: (i, 0)),
            pl.BlockSpec((_BM, _L), lambda i: (i, 0)),
            pl.BlockSpec((_BM, _CP), lambda i: (i, 0)),
            pl.BlockSpec((_D, _D), lambda i: (0, 0)),
            pl.BlockSpec((_D, _D), lambda i: (0, 0)),
            pl.BlockSpec((1, _D), lambda i: (0, 0)),
            pl.BlockSpec((_D, _D), lambda i: (0, 0)),
            pl.BlockSpec((1, _D), lambda i: (0, 0)),
            pl.BlockSpec((_D, _CP), lambda i: (0, 0)),
            pl.BlockSpec((1, _CP), lambda i: (0, 0)),
        ],
        out_specs=[
            pl.BlockSpec((1, 1), lambda i: (0, 0)),
            pl.BlockSpec((_BM, _CP), lambda i: (i, 0)),
        ],
        out_shape=[
            jax.ShapeDtypeStruct((1, 1), jnp.float32),
            jax.ShapeDtypeStruct((_N, _CP), jnp.int32),
        ],
    )(h1, agg2, deg16, tgt128, W_self2, W_neigh2, b2,
      W_lin1, b_lin1, W_lin2p, b_lin2p)


def kernel(x, edge_index, tgt_tags,
           W_self1, W_neigh1, b1,
           W_self2, W_neigh2, b2,
           W_lin1, b_lin1, W_lin2, b_lin2):
    srcl, dstl, cnt, deg_flat = _sc_partition(edge_index)
    deg16 = deg_flat.reshape(_NPAD, _L)

    agg1 = _sc_agg(x, srcl, dstl, cnt).reshape(_NPAD, _D)
    h1 = _tc1(x, agg1, deg16, W_self1, W_neigh1, b1.reshape(1, _D))
    agg2 = _sc_agg(h1, srcl, dstl, cnt).reshape(_NPAD, _D)

    W_lin2p = jnp.concatenate(
        [W_lin2, jnp.zeros((_D, _CP - _C), jnp.float32)], axis=1)
    b_lin2p = jnp.concatenate(
        [b_lin2, jnp.full((_CP - _C,), -1e9, jnp.float32)]).reshape(1, _CP)
    tgt128 = jnp.broadcast_to(tgt_tags[:, None], (_N, _CP))

    loss11, pred128 = _tc2(h1, agg2, deg16, tgt128,
                           W_self2, W_neigh2, b2.reshape(1, _D),
                           W_lin1, b_lin1.reshape(1, _D), W_lin2p, b_lin2p)
    loss = loss11[0, 0] / jnp.float32(_N)
    pred = pred128[:, 0]
    return (loss, pred)


# speedup vs baseline: 2.3885x; 1.1089x over previous
import jax
import jax.numpy as jnp
from jax import lax
from jax.experimental import pallas as pl
from jax.experimental.pallas import tpu as pltpu
from jax.experimental.pallas import tpu_sc as plsc

_N = 10000
_E = 160000
_D = 256
_C = 17
_CP = 128
_NC = 2
_NS = 16
_NT = _NC * _NS
_R = 320
_NPAD = _NT * _R
_CAP = 6400
_SCAN = 4000
_K = 64
_L = 16

_BM = 400
_GRID = _N // _BM

_mesh = plsc.VectorSubcoreMesh(
    core_axis_name="c", subcore_axis_name="s", num_cores=_NC, num_subcores=_NS
)


def _wid():
    return lax.axis_index("s") * _NC + lax.axis_index("c")


def _sc_partition_body(edges_hbm, srcl_hbm, dstl_hbm, cnt_hbm, deg_hbm,
                       srcl_v, dstl_v, cs_a, cd_a, cs_b, cd_b, cnt_v, deg_v,
                       s0, s1, s2, s3):
    wid = _wid()
    lo = wid * _R
    _NCH = _E // _SCAN

    def prefill(i, c):
        srcl_v[pl.ds(i * _L, _L)] = jnp.zeros((_L,), jnp.int32)
        dstl_v[pl.ds(i * _L, _L)] = jnp.full((_L,), _R, jnp.int32)
        return c
    lax.fori_loop(0, (_CAP + _L) // _L, prefill, 0)

    def zdeg(i, c):
        deg_v[pl.ds(i * _L, _L)] = jnp.zeros((_L,), jnp.float32)
        return c
    lax.fori_loop(0, (_R * _L + _L) // _L, zdeg, 0)

    def startc(c, cs, cd, ss, sd):
        pltpu.async_copy(edges_hbm.at[pl.ds(c * _SCAN, _SCAN)], cs, ss)
        pltpu.async_copy(edges_hbm.at[pl.ds(_E + c * _SCAN, _SCAN)], cd, sd)

    def waitc(c, cs, cd, ss, sd):
        pltpu.make_async_copy(
            edges_hbm.at[pl.ds(c * _SCAN, _SCAN)], cs, ss).wait()
        pltpu.make_async_copy(
            edges_hbm.at[pl.ds(_E + c * _SCAN, _SCAN)], cd, sd).wait()

    def scan_chunk(cs, cd, cntv):
        @plsc.parallel_loop(0, _SCAN // _L, carry=cntv, unroll=4)
        def body(g, out):
            lane = lax.iota(jnp.int32, _L)
            ones16 = jnp.ones((_L,), jnp.float32)
            s16 = cs[pl.ds(g * _L, _L)]
            d16 = cd[pl.ds(g * _L, _L)]
            dl = d16 - lo
            m = (dl >= 0) & (dl < _R)
            cum = plsc.cumsum(jnp.where(m, 1, 0))
            pos = jnp.where(m, jnp.minimum(out, _CAP - _L) + cum - 1, _CAP)
            plsc.store_scatter(srcl_v, [pos], s16)
            plsc.store_scatter(dstl_v, [pos], jnp.where(m, dl, _R))
            degidx = jnp.where(m, dl * _L + lane, _R * _L)
            plsc.addupdate_scatter(deg_v, [degidx], ones16)
            return out + plsc.all_reduce_population_count(m)
        return body

    startc(0, cs_a, cd_a, s0, s1)

    def pair(p, cntv):
        c0 = p * 2
        startc(c0 + 1, cs_b, cd_b, s2, s3)
        waitc(c0, cs_a, cd_a, s0, s1)
        cntv = scan_chunk(cs_a, cd_a, cntv)

        @pl.when(c0 + 2 < _NCH)
        def _():
            startc(c0 + 2, cs_a, cd_a, s0, s1)
        waitc(c0 + 1, cs_b, cd_b, s2, s3)
        cntv = scan_chunk(cs_b, cd_b, cntv)
        return cntv
    cntv = lax.fori_loop(0, _NCH // 2, pair, jnp.zeros((_L,), jnp.int32))
    cnt_v[...] = cntv
    pltpu.sync_copy(srcl_v.at[pl.ds(0, _CAP)], srcl_hbm.at[wid])
    pltpu.sync_copy(dstl_v.at[pl.ds(0, _CAP)], dstl_hbm.at[wid])
    pltpu.sync_copy(cnt_v, cnt_hbm.at[wid])
    pltpu.sync_copy(deg_v.at[pl.ds(0, _R * _L)],
                    deg_hbm.at[pl.ds(wid * _R * _L, _R * _L)])


def _sc_partition(edge_index):
    f = pl.kernel(
        _sc_partition_body,
        out_type=[
            jax.ShapeDtypeStruct((_NT, _CAP), jnp.int32),
            jax.ShapeDtypeStruct((_NT, _CAP), jnp.int32),
            jax.ShapeDtypeStruct((_NT, _L), jnp.int32),
            jax.ShapeDtypeStruct((_NPAD * _L,), jnp.float32),
        ],
        mesh=_mesh,
        compiler_params=pltpu.CompilerParams(needs_layout_passes=False),
        scratch_types=[
            pltpu.VMEM((_CAP + _L,), jnp.int32),
            pltpu.VMEM((_CAP + _L,), jnp.int32),
            pltpu.VMEM((_SCAN,), jnp.int32),
            pltpu.VMEM((_SCAN,), jnp.int32),
            pltpu.VMEM((_SCAN,), jnp.int32),
            pltpu.VMEM((_SCAN,), jnp.int32),
            pltpu.VMEM((_L,), jnp.int32),
            pltpu.VMEM((_R * _L + _L,), jnp.float32),
            pltpu.SemaphoreType.DMA,
            pltpu.SemaphoreType.DMA,
            pltpu.SemaphoreType.DMA,
            pltpu.SemaphoreType.DMA,
        ],
    )
    return f(edge_index.reshape(2 * _E))


def _sc_agg_body(z_hbm, srcl_hbm, dstl_hbm, cnt_hbm, agg_hbm,
                 srcl_v, dstl_v, cnt_v, acc_v, rows0, rows1, sem0, sem1):
    wid = _wid()
    lane = lax.iota(jnp.int32, _L)
    pltpu.sync_copy(srcl_hbm.at[wid], srcl_v)
    pltpu.sync_copy(dstl_hbm.at[wid], dstl_v)
    pltpu.sync_copy(cnt_hbm.at[wid], cnt_v)

    zf16 = jnp.zeros((_L,), jnp.float32)

    def zacc(i, c):
        acc_v[pl.ds(i * _L, _L)] = zf16
        return c
    lax.fori_loop(0, (_R + 1) * _D // _L, zacc, 0)

    n = jnp.max(cnt_v[...])
    n = jnp.minimum(jnp.maximum(n, 1), _CAP)
    nch = (n + _K - 1) // _K

    def start(ci, rows, sem):
        pltpu.async_copy(z_hbm.at[srcl_v.at[pl.ds(ci * _K, _K)]], rows, sem)

    def wait(ci, rows, sem):
        pltpu.make_async_copy(
            z_hbm.at[srcl_v.at[pl.ds(ci * _K, _K)]], rows, sem).wait()

    def process(ebase, rows):
        @plsc.parallel_loop(0, _K // _L, unroll=2)
        def grp(q):
            lane_i = lax.iota(jnp.int32, _L)
            off16 = dstl_v[pl.ds(ebase + q * _L, _L)] * _D
            for k in range(_L):
                off = jnp.sum(jnp.where(lane_i == k, off16, 0))
                e = q * _L + k
                for j in range(_D // _L):
                    plsc.addupdate(acc_v.at[pl.ds(off + j * _L, _L)],
                                   rows[e, pl.ds(j * _L, _L)])

    start(0, rows0, sem0)
    npairs = (nch + 1) // 2

    def pair(p, c):
        c0 = p * 2

        @pl.when(c0 + 1 < nch)
        def _():
            start(c0 + 1, rows1, sem1)
        wait(c0, rows0, sem0)
        process(c0 * _K, rows0)

        @pl.when(c0 + 2 < nch)
        def _():
            start(c0 + 2, rows0, sem0)

        @pl.when(c0 + 1 < nch)
        def _():
            wait(c0 + 1, rows1, sem1)
            process((c0 + 1) * _K, rows1)
        return c
    lax.fori_loop(0, npairs, pair, 0)
    pltpu.sync_copy(acc_v.at[pl.ds(0, _R * _D)],
                    agg_hbm.at[pl.ds(wid * _R * _D, _R * _D)])


def _sc_agg(z, srcl, dstl, cnt):
    f = pl.kernel(
        _sc_agg_body,
        out_type=jax.ShapeDtypeStruct((_NPAD * _D,), jnp.float32),
        mesh=_mesh,
        compiler_params=pltpu.CompilerParams(needs_layout_passes=False),
        scratch_types=[
            pltpu.VMEM((_CAP,), jnp.int32),
            pltpu.VMEM((_CAP,), jnp.int32),
            pltpu.VMEM((_L,), jnp.int32),
            pltpu.VMEM(((_R + 1) * _D,), jnp.float32),
            pltpu.VMEM((_K, _D), jnp.float32),
            pltpu.VMEM((_K, _D), jnp.float32),
            pltpu.SemaphoreType.DMA,
            pltpu.SemaphoreType.DMA,
        ],
    )
    return f(z, srcl, dstl, cnt)


def _elu(v):
    return jnp.where(v > 0, v, jnp.exp(jnp.minimum(v, 0.0)) - 1.0)


def _tc1_body(x_ref, agg_ref, deg_ref, ws_ref, wn_ref, b_ref, h_ref):
    deg = jnp.sum(deg_ref[...], axis=1, keepdims=True)
    degm = jnp.maximum(deg, 1.0)
    aggd = agg_ref[...] / degm
    h_ref[...] = _elu(
        jnp.dot(x_ref[...], ws_ref[...], preferred_element_type=jnp.float32)
        + jnp.dot(aggd, wn_ref[...], preferred_element_type=jnp.float32)
        + b_ref[...])


def _tc1(x, agg1, deg16, W_self1, W_neigh1, b1):
    return pl.pallas_call(
        _tc1_body,
        grid=(_GRID,),
        in_specs=[
            pl.BlockSpec((_BM, _D), lambda i: (i, 0)),
            pl.BlockSpec((_BM, _D), lambda i: (i, 0)),
            pl.BlockSpec((_BM, _L), lambda i: (i, 0)),
            pl.BlockSpec((_D, _D), lambda i: (0, 0)),
            pl.BlockSpec((_D, _D), lambda i: (0, 0)),
            pl.BlockSpec((1, _D), lambda i: (0, 0)),
        ],
        out_specs=pl.BlockSpec((_BM, _D), lambda i: (i, 0)),
        out_shape=jax.ShapeDtypeStruct((_N, _D), jnp.float32),
    )(x, agg1, deg16, W_self1, W_neigh1, b1)


def _tc2_body(h1_ref, agg_ref, deg_ref, tgt_ref, ws_ref, wn_ref, b_ref,
              wl1_ref, bl1_ref, wl2_ref, bl2_ref, loss_ref, pred_ref):
    i = pl.program_id(0)
    deg = jnp.sum(deg_ref[...], axis=1, keepdims=True)
    degm = jnp.maximum(deg, 1.0)
    aggd = agg_ref[...] / degm
    h2 = _elu(
        jnp.dot(h1_ref[...], ws_ref[...], preferred_element_type=jnp.float32)
        + jnp.dot(aggd, wn_ref[...], preferred_element_type=jnp.float32)
        + b_ref[...])
    h3 = _elu(jnp.dot(h2, wl1_ref[...], preferred_element_type=jnp.float32)
              + bl1_ref[...])
    lg = (jnp.dot(h3, wl2_ref[...], preferred_element_type=jnp.float32)
          + bl2_ref[...])
    m = jnp.max(lg, axis=1, keepdims=True)
    ex = jnp.exp(lg - m)
    lse = m + jnp.log(jnp.sum(ex, axis=1, keepdims=True))
    lanes = lax.broadcasted_iota(jnp.int32, (_BM, _CP), 1)
    picked = jnp.sum(jnp.where(tgt_ref[...] == lanes, lg, 0.0),
                     axis=1, keepdims=True)
    part = jnp.reshape(jnp.sum(lse - picked), (1, 1))

    @pl.when(i == 0)
    def _():
        loss_ref[...] = jnp.zeros((1, 1), jnp.float32)
    loss_ref[...] += part
    ismax = lg == m
    idx = jnp.min(jnp.where(ismax, lanes, _CP), axis=1, keepdims=True)
    pred_ref[...] = jnp.broadcast_to(idx, (_BM, _CP))


def _tc2(h1, agg2, deg16, tgt128, W_self2, W_neigh2, b2,
         W_lin1, b_lin1, W_lin2p, b_lin2p):
    return pl.pallas_call(
        _tc2_body,
        grid=(_GRID,),
        in_specs=[
            pl.BlockSpec((_BM, _D), lambda i: (i, 0)),
            pl.BlockSpec((_BM, _D), lambda i: (i, 0)),
            pl.BlockSpec((_BM, _L), lambda i: (i, 0)),
            pl.BlockSpec((_BM, _CP), lambda i: (i, 0)),
            pl.BlockSpec((_D, _D), lambda i: (0, 0)),
            pl.BlockSpec((_D, _D), lambda i: (0, 0)),
            pl.BlockSpec((1, _D), lambda i: (0, 0)),
            pl.BlockSpec((_D, _D), lambda i: (0, 0)),
            pl.BlockSpec((1, _D), lambda i: (0, 0)),
            pl.BlockSpec((_D, _CP), lambda i: (0, 0)),
            pl.BlockSpec((1, _CP), lambda i: (0, 0)),
        ],
        out_specs=[
            pl.BlockSpec((1, 1), lambda i: (0, 0)),
            pl.BlockSpec((_BM, _CP), lambda i: (i, 0)),
        ],
        out_shape=[
            jax.ShapeDtypeStruct((1, 1), jnp.float32),
            jax.ShapeDtypeStruct((_N, _CP), jnp.int32),
        ],
    )(h1, agg2, deg16, tgt128, W_self2, W_neigh2, b2,
      W_lin1, b_lin1, W_lin2p, b_lin2p)


def kernel(x, edge_index, tgt_tags,
           W_self1, W_neigh1, b1,
           W_self2, W_neigh2, b2,
           W_lin1, b_lin1, W_lin2, b_lin2):
    srcl, dstl, cnt, deg_flat = _sc_partition(edge_index)
    deg16 = deg_flat.reshape(_NPAD, _L)

    agg1 = _sc_agg(x, srcl, dstl, cnt).reshape(_NPAD, _D)
    h1 = _tc1(x, agg1, deg16, W_self1, W_neigh1, b1.reshape(1, _D))
    agg2 = _sc_agg(h1, srcl, dstl, cnt).reshape(_NPAD, _D)

    W_lin2p = jnp.concatenate(
        [W_lin2, jnp.zeros((_D, _CP - _C), jnp.float32)], axis=1)
    b_lin2p = jnp.concatenate(
        [b_lin2, jnp.full((_CP - _C,), -1e9, jnp.float32)]).reshape(1, _CP)
    tgt128 = jnp.broadcast_to(tgt_tags[:, None], (_N, _CP))

    loss11, pred128 = _tc2(h1, agg2, deg16, tgt128,
                           W_self2, W_neigh2, b2.reshape(1, _D),
                           W_lin1, b_lin1.reshape(1, _D), W_lin2p, b_lin2p)
    loss = loss11[0, 0] / jnp.float32(_N)
    pred = pred128[:, 0]
    return (loss, pred)
